# baseline (speedup 1.0000x reference)
import jax
import jax.numpy as jnp
from jax import lax
from jax.experimental import pallas as pl
from jax.experimental.pallas import tpu as pltpu
from jax.experimental.pallas import tpu_sc as plsc

NC = 2
NS = 16
NW = NC * NS
L = 16

B = 4096
F_NUM = 13
F_CAT = 26
CARD1 = 1001
D = 128
T_TOK = F_NUM + F_CAT

B_PER_W = B // NW
E_PER_W = B_PER_W * F_CAT
NBUF = 6
LOOKAHEAD = 4


def _gather_body(tab_hbm, xcat_hbm, out_hbm, idx_v, bufs, gsem, ssem):
    wid = lax.axis_index("s") * NC + lax.axis_index("c")
    e0 = wid * E_PER_W

    pltpu.sync_copy(xcat_hbm.at[pl.ds(wid, 1)], idx_v.reshape(1, F_CAT, D))

    iota = lax.iota(jnp.int32, L)

    def _gidx_body(p, carry):
        base = p * D
        for v in range(D // L):
            jv = iota + (base + v * L)
            r = lax.rem(jv, F_CAT)
            sl = pl.ds(v * L, L)
            idx_v[p, sl] = idx_v[p, sl] + r * CARD1 + 1
        return carry

    lax.fori_loop(0, F_CAT, _gidx_body, 0)

    def _fire_gather(p):
        j = p % NBUF
        return pltpu.async_copy(tab_hbm.at[idx_v.at[p]], bufs[j], gsem[j])

    def _fire_store(p):
        j = p % NBUF
        return pltpu.async_copy(
            bufs[j], out_hbm.at[pl.ds(e0 + p * D, D)], ssem[j])

    gh = [None] * F_CAT
    sh = [None] * F_CAT
    for p in range(min(LOOKAHEAD, F_CAT)):
        gh[p] = _fire_gather(p)
    for p in range(F_CAT):
        gh[p].wait()
        sh[p] = _fire_store(p)
        q = p + LOOKAHEAD
        if q < F_CAT:
            d = q - NBUF
            if d >= 0:
                sh[d].wait()
            gh[q] = _fire_gather(q)
    for p in range(F_CAT - NBUF, F_CAT):
        if p >= 0 and sh[p] is not None:
            sh[p].wait()


def _build_sc_call():
    mesh = plsc.VectorSubcoreMesh(
        core_axis_name="c", subcore_axis_name="s",
        num_cores=NC, num_subcores=NS)
    scratch = [
        pltpu.VMEM((F_CAT, D), jnp.int32),
        [pltpu.VMEM((D, D), jnp.float32) for _ in range(NBUF)],
        [pltpu.SemaphoreType.DMA for _ in range(NBUF)],
        [pltpu.SemaphoreType.DMA for _ in range(NBUF)],
    ]
    return pl.kernel(
        _gather_body,
        out_type=jax.ShapeDtypeStruct((B * F_CAT, D), jnp.float32),
        mesh=mesh,
        scratch_types=scratch,
        name="feature_tokenizer_sc_gather",
    )


_SC_CALL = _build_sc_call()

NB = 32


def _tc_body(cat_ref, xn_ref, w_ref, b_ref, out_ref):
    cat = cat_ref[...].reshape(NB, F_CAT, D)
    num = (xn_ref[...][:, :, None] * w_ref[...][None, :, :]
           + b_ref[...][None, :, :])
    out_ref[...] = jnp.concatenate([num, cat], axis=1)


_TC_CALL = pl.pallas_call(
    _tc_body,
    grid=(B // NB,),
    in_specs=[
        pl.BlockSpec((NB * F_CAT, D), lambda i: (i, 0)),
        pl.BlockSpec((NB, F_NUM), lambda i: (i, 0)),
        pl.BlockSpec((F_NUM, D), lambda i: (0, 0)),
        pl.BlockSpec((F_NUM, D), lambda i: (0, 0)),
    ],
    out_specs=pl.BlockSpec((NB, T_TOK, D), lambda i: (i, 0, 0)),
    out_shape=jax.ShapeDtypeStruct((B, T_TOK, D), jnp.float32),
)


def kernel(x_cat, x_num, W_num, b_num, tables):
    xcat3 = x_cat.astype(jnp.int32).reshape(NW, F_CAT, D)
    tab = tables.reshape(F_CAT * CARD1, D)
    cat_rows = _SC_CALL(tab, xcat3)
    return _TC_CALL(cat_rows, x_num, W_num, b_num)
